# Initial kernel scaffold; baseline (speedup 1.0000x reference)
import functools

import jax
import jax.numpy as jnp
from jax import lax
from jax.experimental import pallas as pl
from jax.experimental.pallas import tpu as pltpu
from jax.experimental.pallas import tpu_sc as plsc

B = 16384
L = 200
EMB = 32
N = B * L


def _build_gather(n_rows: int, emb: int, chunk: int):
    info = plsc.get_sparse_core_info()
    nc, ns = info.num_cores, info.num_subcores
    nw = nc * ns
    per_w = n_rows // nw
    n_chunks = per_w // chunk
    assert per_w % chunk == 0 and n_rows % nw == 0

    mesh = plsc.VectorSubcoreMesh(core_axis_name="c", subcore_axis_name="s")

    @functools.partial(
        pl.kernel,
        mesh=mesh,
        out_type=jax.ShapeDtypeStruct((n_rows, emb), jnp.float32),
        scratch_types=[
            pltpu.VMEM((chunk,), jnp.int32),
            pltpu.VMEM((chunk, emb), jnp.float32),
            pltpu.SemaphoreType.DMA,
        ],
    )
    def gather_kernel(table_hbm, idx_hbm, out_hbm, idx_v, rows_v, sem):
        wid = lax.axis_index("s") * nc + lax.axis_index("c")
        base = pl.multiple_of(wid * per_w, chunk)

        def body(i, carry):
            off = pl.multiple_of(base + i * chunk, chunk)
            pltpu.sync_copy(idx_hbm.at[pl.ds(off, chunk)], idx_v)
            pltpu.async_copy(table_hbm.at[idx_v], rows_v, sem).wait()
            pltpu.sync_copy(rows_v, out_hbm.at[pl.ds(off, chunk)])
            return carry

        lax.fori_loop(0, n_chunks, body, 0)

    return gather_kernel


def kernel(prev_actions, table):
    idx = prev_actions.reshape(N).astype(jnp.int32)
    out = _build_gather(N, EMB, chunk=1600)(table, idx)
    return out.reshape(B, L, EMB)


# speedup vs baseline: 4.9059x; 4.9059x over previous
import functools

import jax
import jax.numpy as jnp
from jax import lax
from jax.experimental import pallas as pl
from jax.experimental.pallas import tpu as pltpu
from jax.experimental.pallas import tpu_sc as plsc

B = 16384
L = 200
EMB = 32
N = B * L


def _build_gather(n_rows: int, emb: int, chunk: int):
    info = plsc.get_sparse_core_info()
    nc, ns = info.num_cores, info.num_subcores
    nw = nc * ns
    per_w = n_rows // nw
    n_chunks = per_w // chunk
    assert per_w % chunk == 0 and n_rows % nw == 0

    mesh = plsc.VectorSubcoreMesh(core_axis_name="c", subcore_axis_name="s")

    @functools.partial(
        pl.kernel,
        mesh=mesh,
        compiler_params=pltpu.CompilerParams(use_tc_tiling_on_sc=False),
        out_type=jax.ShapeDtypeStruct((n_rows, emb), jnp.float32),
        scratch_types=[
            pltpu.VMEM((chunk,), jnp.int32),
            pltpu.VMEM((chunk, emb), jnp.float32),
            pltpu.SemaphoreType.DMA,
        ],
    )
    def gather_kernel(table_hbm, idx_hbm, out_hbm, idx_v, rows_v, sem):
        wid = lax.axis_index("s") * nc + lax.axis_index("c")
        base = pl.multiple_of(wid * per_w, chunk)

        def body(i, carry):
            off = pl.multiple_of(base + i * chunk, chunk)
            pltpu.sync_copy(idx_hbm.at[pl.ds(off, chunk)], idx_v)
            pltpu.async_copy(table_hbm.at[idx_v], rows_v, sem).wait()
            pltpu.sync_copy(rows_v, out_hbm.at[pl.ds(off, chunk)])
            return carry

        lax.fori_loop(0, n_chunks, body, 0)

    return gather_kernel


def kernel(prev_actions, table):
    idx = prev_actions.reshape(N).astype(jnp.int32)
    out = _build_gather(N, EMB, chunk=1600)(table, idx)
    return out.reshape(B, L, EMB)


# speedup vs baseline: 4.9756x; 1.0142x over previous
import functools

import jax
import jax.numpy as jnp
from jax import lax
from jax.experimental import pallas as pl
from jax.experimental.pallas import tpu as pltpu
from jax.experimental.pallas import tpu_sc as plsc

B = 16384
L = 200
EMB = 32
N = B * L


def _build_gather(n_rows: int, emb: int, chunk: int):
    info = plsc.get_sparse_core_info()
    nc, ns = info.num_cores, info.num_subcores
    nw = nc * ns
    per_w = n_rows // nw
    n_chunks = per_w // chunk
    assert per_w % chunk == 0 and n_rows % nw == 0
    assert n_chunks % 2 == 0

    mesh = plsc.VectorSubcoreMesh(core_axis_name="c", subcore_axis_name="s")

    @functools.partial(
        pl.kernel,
        mesh=mesh,
        compiler_params=pltpu.CompilerParams(use_tc_tiling_on_sc=False),
        out_type=jax.ShapeDtypeStruct((n_rows, emb), jnp.float32),
        scratch_types=[
            pltpu.VMEM((2, chunk), jnp.int32),
            pltpu.VMEM((2, chunk, emb), jnp.float32),
            pltpu.SemaphoreType.DMA,
            pltpu.SemaphoreType.DMA,
        ],
    )
    def gather_kernel(table_hbm, idx_hbm, out_hbm, idx_v, rows_v, sem0, sem1):
        sems = (sem0, sem1)
        wid = lax.axis_index("s") * nc + lax.axis_index("c")
        base = pl.multiple_of(wid * per_w, chunk)

        pltpu.sync_copy(idx_hbm.at[pl.ds(base, chunk)], idx_v.at[0])
        pltpu.async_copy(table_hbm.at[idx_v.at[0]], rows_v.at[0], sems[0])

        def outer(j, carry):
            for b in range(2):
                i = j * 2 + b
                nxt = jnp.minimum(i + 1, n_chunks - 1)
                noff = pl.multiple_of(base + nxt * chunk, 8)
                pltpu.sync_copy(idx_hbm.at[pl.ds(noff, chunk)],
                                idx_v.at[1 - b])
                pltpu.async_copy(table_hbm.at[idx_v.at[1 - b]],
                                 rows_v.at[1 - b], sems[1 - b])
                pltpu.make_async_copy(table_hbm.at[idx_v.at[b]],
                                      rows_v.at[b], sems[b]).wait()
                off = pl.multiple_of(base + i * chunk, chunk)
                pltpu.sync_copy(rows_v.at[b], out_hbm.at[pl.ds(off, chunk)])
            return carry

        lax.fori_loop(0, n_chunks // 2, outer, 0)
        pltpu.make_async_copy(table_hbm.at[idx_v.at[0]], rows_v.at[0],
                              sems[0]).wait()

    return gather_kernel


def kernel(prev_actions, table):
    idx = prev_actions.reshape(N).astype(jnp.int32)
    out = _build_gather(N, EMB, chunk=1600)(table, idx)
    return out.reshape(B, L, EMB)
